# baseline (speedup 1.0000x reference)
import functools

import jax
import jax.numpy as jnp
import numpy as np
from jax import lax
from jax.experimental import pallas as pl
from jax.experimental.pallas import tpu as pltpu
from jax.experimental.pallas import tpu_sc as plsc

_C = 128
_K = 32
_N = 10000
_NW = 32
_NPW = 320
_NPAD = _NW * _NPW
_CW = _C // 2

_CH = 4
_NCH = _NPW // _CH
_RK = _CH * _K

_PERM = np.empty(_C, dtype=np.int32)
_PERM[0::2] = np.arange(_CW)
_PERM[1::2] = _CW + np.arange(_CW)


def _tc_tables_body(x_ref, w_ref, b_ref, gaL_ref, gaR_ref, gbL_ref, gbR_ref):
    x = x_ref[...]
    w1 = w_ref[:, :_C]
    w2 = w_ref[:, _C:]
    dn = (((0,), (1,)), ((), ()))
    ga = lax.dot_general(x, w1 - w2, dn,
                         preferred_element_type=jnp.float32,
                         precision=lax.Precision.HIGHEST)
    gb = lax.dot_general(x, w2, dn,
                         preferred_element_type=jnp.float32,
                         precision=lax.Precision.HIGHEST)
    ga_bf = (ga + b_ref[...][None, :]).astype(jnp.bfloat16)
    gb_bf = gb.astype(jnp.bfloat16)
    gaL_ref[...] = ga_bf[:, :_C // 2]
    gaR_ref[...] = ga_bf[:, _C // 2:]
    gbL_ref[...] = gb_bf[:, :_C // 2]
    gbR_ref[...] = gb_bf[:, _C // 2:]


def _tc_tables(x2d, Wp, bp):
    return pl.pallas_call(
        _tc_tables_body,
        out_shape=tuple(
            jax.ShapeDtypeStruct((_N, _C // 2), jnp.bfloat16) for _ in range(4)
        ),
    )(x2d, Wp, bp)


_HW = _CW // 2


def _sc_body(gaL, gaR, gbL, gbR, idxa, idxb, out,
             idxa_v, idxb_v, aL0, aR0, bL0, bR0, aL1, aR1, bL1, bR1, out_v,
             s00, s01, s02, s03, s10, s11, s12, s13):
    wid = lax.axis_index("s") * 2 + lax.axis_index("c")
    base = wid * _NPW
    cbase = wid * _NCH
    pltpu.sync_copy(idxa.at[pl.ds(cbase, _NCH)], idxa_v)
    pltpu.sync_copy(idxb.at[pl.ds(cbase, _NCH)], idxb_v)

    slots = (((aL0, aR0, bL0, bR0), (s00, s01, s02, s03)),
             ((aL1, aR1, bL1, bR1), (s10, s11, s12, s13)))
    tables = (gaL, gaR, gbL, gbR)

    half = _RK // 2

    def issue(g, slot):
        bufs, sems = slot
        for t in range(4):
            iv = idxa_v if t < 2 else idxb_v
            pltpu.async_copy(tables[t].at[iv.at[g, pl.ds(0, half)]],
                             bufs[t].at[pl.ds(0, half)], sems[t])
            pltpu.async_copy(tables[t].at[iv.at[g, pl.ds(half, half)]],
                             bufs[t].at[pl.ds(half, half)], sems[t])

    def wait(slot):
        bufs, sems = slot
        for t in range(4):
            pltpu.make_async_copy(tables[t].at[pl.ds(0, half)],
                                  bufs[t].at[pl.ds(0, half)], sems[t]).wait()
            pltpu.make_async_copy(tables[t].at[pl.ds(0, half)],
                                  bufs[t].at[pl.ds(half, half)], sems[t]).wait()

    himask = jnp.full((16,), -65536, jnp.int32)

    def halves(w):
        lo = lax.bitcast_convert_type(lax.shift_left(w, 16), jnp.float32)
        hi = lax.bitcast_convert_type(lax.bitwise_and(w, himask), jnp.float32)
        return lo, hi

    def compute(g, slot):
        bufs, _ = slot
        zero = jnp.zeros((16,), jnp.float32)
        for ni in range(_CH):
            n = g * _CH + ni
            r = ni * _K
            for c in range(_CW // 16):
                abuf = bufs[0] if c < 2 else bufs[1]
                bbuf = bufs[2] if c < 2 else bufs[3]
                s = pl.ds((c % 2) * 16, 16)
                alo, ahi = halves(abuf[r, s])
                blo, bhi = halves(bbuf[r, s])
                acc_lo = alo + blo
                acc_hi = ahi + bhi
                for k in range(1, _K):
                    alo, ahi = halves(abuf[r + k, s])
                    blo, bhi = halves(bbuf[r + k, s])
                    acc_lo = jnp.maximum(acc_lo, alo + blo)
                    acc_hi = jnp.maximum(acc_hi, ahi + bhi)
                out_v[n, pl.ds(c * 16, 16)] = jnp.maximum(acc_lo, zero)
                out_v[n, pl.ds(_CW + c * 16, 16)] = jnp.maximum(acc_hi, zero)

    issue(0, slots[0])
    issue(1, slots[1])

    def pair_body(t, carry):
        for p in range(2):
            g = 2 * t + p
            wait(slots[p])
            compute(g, slots[p])

            @pl.when(g + 2 < _NCH)
            def _():
                issue(g + 2, slots[p])
        return carry

    lax.fori_loop(0, _NCH // 2, pair_body, 0)
    pltpu.sync_copy(out_v, out.at[pl.ds(base, _NPW)])


@functools.cache
def _sc_gather_max():
    return pl.kernel(
        _sc_body,
        out_type=jax.ShapeDtypeStruct((_NPAD, _C), jnp.float32),
        mesh=plsc.VectorSubcoreMesh(core_axis_name="c", subcore_axis_name="s"),
        compiler_params=pltpu.CompilerParams(use_tc_tiling_on_sc=False),
        scratch_types=[
            pltpu.VMEM((_NCH, _CH * _K), jnp.int32),
            pltpu.VMEM((_NCH, _CH * _K), jnp.int32),
            pltpu.VMEM((_RK, _HW), jnp.int32),
            pltpu.VMEM((_RK, _HW), jnp.int32),
            pltpu.VMEM((_RK, _HW), jnp.int32),
            pltpu.VMEM((_RK, _HW), jnp.int32),
            pltpu.VMEM((_RK, _HW), jnp.int32),
            pltpu.VMEM((_RK, _HW), jnp.int32),
            pltpu.VMEM((_RK, _HW), jnp.int32),
            pltpu.VMEM((_RK, _HW), jnp.int32),
            pltpu.VMEM((_NPW, _C), jnp.float32),
            pltpu.SemaphoreType.DMA,
            pltpu.SemaphoreType.DMA,
            pltpu.SemaphoreType.DMA,
            pltpu.SemaphoreType.DMA,
            pltpu.SemaphoreType.DMA,
            pltpu.SemaphoreType.DMA,
            pltpu.SemaphoreType.DMA,
            pltpu.SemaphoreType.DMA,
        ],
    )


def kernel(x, edge_index, W, b):
    B = x.shape[0]
    x2d = x.reshape(_C, _N)
    idx0 = edge_index[0].reshape(_N, _K).astype(jnp.int32)
    idx1 = edge_index[1].reshape(_N, _K).astype(jnp.int32)
    pad = ((0, _NPAD - _N), (0, 0))
    idxa = jnp.pad(idx1, pad).reshape(_NPAD // _CH, _CH * _K)
    idxb = jnp.pad(idx0, pad).reshape(_NPAD // _CH, _CH * _K)

    perm = jnp.asarray(_PERM)
    tabs = _tc_tables(x2d, W[perm, :], b[perm])
    tabs_w = [lax.bitcast_convert_type(t.reshape(_N, _HW, 2), jnp.int32)
              for t in tabs]
    out_rows = _sc_gather_max()(*tabs_w, idxa, idxb)
    return out_rows[:_N].T.reshape(B, _C, _N, 1)


# speedup vs baseline: 1.0220x; 1.0220x over previous
import functools

import jax
import jax.numpy as jnp
import numpy as np
from jax import lax
from jax.experimental import pallas as pl
from jax.experimental.pallas import tpu as pltpu
from jax.experimental.pallas import tpu_sc as plsc

_C = 128
_K = 32
_N = 10000
_NW = 32
_NPW = 320
_NPAD = _NW * _NPW
_CW = _C // 2

_CH = 4
_NCH = _NPW // _CH
_RK = _CH * _K

_PERM = np.empty(_C, dtype=np.int32)
_PERM[0::2] = np.arange(_CW)
_PERM[1::2] = _CW + np.arange(_CW)


def _tc_tables_body(x_ref, w_ref, b_ref, gaL_ref, gaR_ref, gbL_ref, gbR_ref):
    x = x_ref[...]
    w1 = w_ref[:, :_C]
    w2 = w_ref[:, _C:]
    dn = (((0,), (1,)), ((), ()))
    ga = lax.dot_general(x, w1 - w2, dn,
                         preferred_element_type=jnp.float32,
                         precision=lax.Precision.HIGHEST)
    gb = lax.dot_general(x, w2, dn,
                         preferred_element_type=jnp.float32,
                         precision=lax.Precision.HIGHEST)
    ga_bf = (ga + b_ref[...][None, :]).astype(jnp.bfloat16)
    gb_bf = gb.astype(jnp.bfloat16)
    gaL_ref[...] = ga_bf[:, :_C // 2]
    gaR_ref[...] = ga_bf[:, _C // 2:]
    gbL_ref[...] = gb_bf[:, :_C // 2]
    gbR_ref[...] = gb_bf[:, _C // 2:]


def _tc_tables(x2d, Wp, bp):
    return pl.pallas_call(
        _tc_tables_body,
        out_shape=tuple(
            jax.ShapeDtypeStruct((_N, _C // 2), jnp.bfloat16) for _ in range(4)
        ),
    )(x2d, Wp, bp)


_HW = _CW // 2


def _sc_body(gaL, gaR, gbL, gbR, idxa, idxb, out,
             idxa_v, idxb_v, aL0, aR0, bL0, bR0, aL1, aR1, bL1, bR1, out_v,
             s00, s01, s02, s03, s10, s11, s12, s13):
    wid = lax.axis_index("s") * 2 + lax.axis_index("c")
    base = wid * _NPW
    cbase = wid * _NCH
    pltpu.sync_copy(idxa.at[pl.ds(cbase, _NCH)], idxa_v)
    pltpu.sync_copy(idxb.at[pl.ds(cbase, _NCH)], idxb_v)

    slots = (((aL0, aR0, bL0, bR0), (s00, s01, s02, s03)),
             ((aL1, aR1, bL1, bR1), (s10, s11, s12, s13)))
    tables = (gaL, gaR, gbL, gbR)

    def issue(g, slot):
        bufs, sems = slot
        for t in range(4):
            iv = idxa_v if t < 2 else idxb_v
            pltpu.async_copy(tables[t].at[iv.at[g]], bufs[t], sems[t])

    def wait(slot):
        bufs, sems = slot
        for t in range(4):
            pltpu.make_async_copy(tables[t].at[pl.ds(0, _RK)],
                                  bufs[t], sems[t]).wait()

    himask = jnp.full((16,), -65536, jnp.int32)

    def halves(w):
        lo = lax.bitcast_convert_type(lax.shift_left(w, 16), jnp.float32)
        hi = lax.bitcast_convert_type(lax.bitwise_and(w, himask), jnp.float32)
        return lo, hi

    def compute(g, slot):
        bufs, _ = slot
        zero = jnp.zeros((16,), jnp.float32)
        for ni in range(_CH):
            n = g * _CH + ni
            r = ni * _K
            for c in range(_CW // 16):
                abuf = bufs[0] if c < 2 else bufs[1]
                bbuf = bufs[2] if c < 2 else bufs[3]
                s = pl.ds((c % 2) * 16, 16)
                alo, ahi = halves(abuf[r, s])
                blo, bhi = halves(bbuf[r, s])
                acc_lo = alo + blo
                acc_hi = ahi + bhi
                for k in range(1, _K):
                    alo, ahi = halves(abuf[r + k, s])
                    blo, bhi = halves(bbuf[r + k, s])
                    acc_lo = jnp.maximum(acc_lo, alo + blo)
                    acc_hi = jnp.maximum(acc_hi, ahi + bhi)
                out_v[n, pl.ds(c * 16, 16)] = jnp.maximum(acc_lo, zero)
                out_v[n, pl.ds(_CW + c * 16, 16)] = jnp.maximum(acc_hi, zero)

    issue(0, slots[0])
    issue(1, slots[1])

    def pair_body(t, carry):
        for p in range(2):
            g = 2 * t + p
            wait(slots[p])
            compute(g, slots[p])

            @pl.when(g + 2 < _NCH)
            def _():
                issue(g + 2, slots[p])
        return carry

    lax.fori_loop(0, _NCH // 2, pair_body, 0)
    pltpu.sync_copy(out_v, out.at[pl.ds(base, _NPW)])


@functools.cache
def _sc_gather_max():
    return pl.kernel(
        _sc_body,
        out_type=jax.ShapeDtypeStruct((_NPAD, _C), jnp.float32),
        mesh=plsc.VectorSubcoreMesh(core_axis_name="c", subcore_axis_name="s"),
        compiler_params=pltpu.CompilerParams(use_tc_tiling_on_sc=False),
        scratch_types=[
            pltpu.VMEM((_NCH, _CH * _K), jnp.int32),
            pltpu.VMEM((_NCH, _CH * _K), jnp.int32),
            pltpu.VMEM((_RK, _HW), jnp.int32),
            pltpu.VMEM((_RK, _HW), jnp.int32),
            pltpu.VMEM((_RK, _HW), jnp.int32),
            pltpu.VMEM((_RK, _HW), jnp.int32),
            pltpu.VMEM((_RK, _HW), jnp.int32),
            pltpu.VMEM((_RK, _HW), jnp.int32),
            pltpu.VMEM((_RK, _HW), jnp.int32),
            pltpu.VMEM((_RK, _HW), jnp.int32),
            pltpu.VMEM((_NPW, _C), jnp.float32),
            pltpu.SemaphoreType.DMA,
            pltpu.SemaphoreType.DMA,
            pltpu.SemaphoreType.DMA,
            pltpu.SemaphoreType.DMA,
            pltpu.SemaphoreType.DMA,
            pltpu.SemaphoreType.DMA,
            pltpu.SemaphoreType.DMA,
            pltpu.SemaphoreType.DMA,
        ],
    )


def kernel(x, edge_index, W, b):
    B = x.shape[0]
    x2d = x.reshape(_C, _N)
    idx0 = edge_index[0].reshape(_N, _K).astype(jnp.int32)
    idx1 = edge_index[1].reshape(_N, _K).astype(jnp.int32)
    pad = ((0, _NPAD - _N), (0, 0))
    idxa = jnp.pad(idx1, pad).reshape(_NPAD // _CH, _CH * _K)
    idxb = jnp.pad(idx0, pad).reshape(_NPAD // _CH, _CH * _K)

    perm = jnp.asarray(_PERM)
    tabs = _tc_tables(x2d, W[perm, :], b[perm])
    tabs_w = [lax.bitcast_convert_type(t.reshape(_N, _HW, 2), jnp.int32)
              for t in tabs]
    out_rows = _sc_gather_max()(*tabs_w, idxa, idxb)
    return out_rows[:_N].T.reshape(B, _C, _N, 1)
